# baseline (speedup 1.0000x reference)
import jax
import jax.numpy as jnp
from jax import lax
from jax.experimental import pallas as pl
from jax.experimental.pallas import tpu as pltpu
from jax.experimental.pallas import tpu_sc as plsc

N_NODES = 100000
D = 128
B = 50000

L = 16
NSUB = 16

CHUNK = 80

N_COPY = N_NODES // CHUNK
COPY_LAST = N_NODES - CHUNK
N_CW = 2 * (NSUB - 1)
COPY_ITERS = -(-N_COPY // N_CW)

N_P2 = B // CHUNK
P2_LAST = B - CHUNK
P2_ITERS = -(-N_P2 // NSUB)
P2_QUADS = P2_ITERS // 4

WCHUNK = 2000
N_WCH = B // WCHUNK
GRP = WCHUNK // L


def _body(x_hbm, idx_hbm, emb_hbm, newemb_hbm, pulled_hbm, src_hbm,
          w_v, idxc0, idxc1, srcc0, srcc1, rows0, rows1,
          ib0, ib1, ib2, ib3, sb0, sb1, sb2, sb3,
          sio0, sio1, sio2, sio3, sg0, sg1, ssc0, ssc1, sp0, sp1):
    c = lax.axis_index("c")
    s = lax.axis_index("s")
    idxc = (idxc0, idxc1)
    srcc = (srcc0, srcc1)
    rows = (rows0, rows1)
    ib = (ib0, ib1, ib2, ib3)
    sb = (sb0, sb1, sb2, sb3)
    sio = (sio0, sio1, sio2, sio3)
    sg = (sg0, sg1)
    ssc = (ssc0, ssc1)
    sp = (sp0, sp1)

    @pl.when(s == 0)
    @jax.named_scope("wpass")
    def _wpass():
        iota = lax.iota(jnp.int32, L)
        pltpu.async_copy(idx_hbm.at[pl.ds(0, WCHUNK)], idxc[0], sio[0])
        for ci in range(N_WCH):
            b = ci % 2
            if ci + 1 < N_WCH:
                pltpu.async_copy(
                    idx_hbm.at[pl.ds((ci + 1) * WCHUNK, WCHUNK)],
                    idxc[1 - b], sio[1 - b])
            pltpu.make_async_copy(
                idx_hbm.at[pl.ds(ci * WCHUNK, WCHUNK)], idxc[b], sio[b]).wait()
            cstart = ci * WCHUNK

            def grp(g, carry, _b=b, _cstart=cstart):
                idxv = idxc[_b][pl.ds(g * L, L)]
                jv = (_cstart + g * L) + iota
                _, keep = plsc.scan_count(idxv)
                plsc.store_scatter(w_v, [idxv], jv, mask=keep)
                return carry

            lax.fori_loop(0, GRP, grp, 0)
        pltpu.async_copy(idx_hbm.at[pl.ds(0, WCHUNK)], idxc[0], sio[0])
        for ci in range(N_WCH):
            b = ci % 2
            if ci + 1 < N_WCH:
                pltpu.async_copy(
                    idx_hbm.at[pl.ds((ci + 1) * WCHUNK, WCHUNK)],
                    idxc[1 - b], sio[1 - b])
            pltpu.make_async_copy(
                idx_hbm.at[pl.ds(ci * WCHUNK, WCHUNK)], idxc[b], sio[b]).wait()
            if ci >= 2:
                pltpu.make_async_copy(
                    srcc[b], src_hbm.at[pl.ds((ci - 2) * WCHUNK, WCHUNK)],
                    ssc[b]).wait()

            def grp2(g, carry, _b=b):
                idxv = idxc[_b][pl.ds(g * L, L)]
                srcc[_b][pl.ds(g * L, L)] = plsc.load_gather(w_v, [idxv])
                return carry

            lax.fori_loop(0, GRP, grp2, 0)
            pltpu.async_copy(
                srcc[b], src_hbm.at[pl.ds(ci * WCHUNK, WCHUNK)], ssc[b])
        for ci in (N_WCH - 2, N_WCH - 1):
            pltpu.make_async_copy(
                srcc[ci % 2], src_hbm.at[pl.ds(ci * WCHUNK, WCHUNK)],
                ssc[ci % 2]).wait()

    @pl.when(s > 0)
    @jax.named_scope("copy")
    def _copy():
        v = c * (NSUB - 1) + (s - 1)

        def cstart_of(j):
            g = jnp.minimum(v + N_CW * j, N_COPY - 1)
            return jnp.minimum(g * CHUNK, COPY_LAST)

        pltpu.async_copy(emb_hbm.at[pl.ds(cstart_of(0), CHUNK)], rows[0], sg[0])

        def it(jp, carry):
            for bb in range(2):
                j = jp * 2 + bb
                st = cstart_of(j)

                @pl.when(j + 1 < COPY_ITERS)
                def _prefetch():
                    @pl.when(j >= 1)
                    def _drain_out():
                        pltpu.make_async_copy(
                            rows[1 - bb],
                            newemb_hbm.at[pl.ds(cstart_of(j - 1), CHUNK)],
                            ssc[1 - bb]).wait()
                    pltpu.async_copy(
                        emb_hbm.at[pl.ds(cstart_of(j + 1), CHUNK)],
                        rows[1 - bb], sg[1 - bb])

                pltpu.make_async_copy(
                    emb_hbm.at[pl.ds(st, CHUNK)], rows[bb], sg[bb]).wait()
                pltpu.async_copy(
                    rows[bb], newemb_hbm.at[pl.ds(st, CHUNK)], ssc[bb])
            return carry

        lax.fori_loop(0, COPY_ITERS // 2, it, 0)
        for jz in (COPY_ITERS - 2, COPY_ITERS - 1):
            pltpu.make_async_copy(
                rows[jz % 2], newemb_hbm.at[pl.ds(cstart_of(jz), CHUNK)],
                ssc[jz % 2]).wait()

    plsc.subcore_barrier()

    def p2start_of(i):
        g = jnp.minimum(s + NSUB * i, N_P2 - 1)
        return g, jnp.minimum(g * CHUNK, P2_LAST)

    scope2 = jax.named_scope("phase2")
    scope2.__enter__()
    _, st0 = p2start_of(0)
    pltpu.async_copy(idx_hbm.at[pl.ds(st0, CHUNK)], ib[0], sio[0])
    pltpu.async_copy(src_hbm.at[pl.ds(st0, CHUNK)], sb[0], sio[0])

    def it2(q, carry):
        for bb in range(4):
            i = q * 4 + bb
            b2 = bb % 2
            g, st = p2start_of(i)

            @pl.when(i + 1 < P2_ITERS)
            def _prefetch():
                _, st1 = p2start_of(i + 1)
                pltpu.async_copy(
                    idx_hbm.at[pl.ds(st1, CHUNK)], ib[(bb + 1) % 4],
                    sio[(bb + 1) % 4])
                pltpu.async_copy(
                    src_hbm.at[pl.ds(st1, CHUNK)], sb[(bb + 1) % 4],
                    sio[(bb + 1) % 4])

            pltpu.make_async_copy(
                idx_hbm.at[pl.ds(st, CHUNK)], ib[bb], sio[bb]).wait()
            pltpu.make_async_copy(
                src_hbm.at[pl.ds(st, CHUNK)], sb[bb], sio[bb]).wait()

            @pl.when(i >= 2)
            def _drain_prev():
                g2, st2 = p2start_of(i - 2)
                pltpu.make_async_copy(
                    rows[b2], newemb_hbm.at[ib[(bb + 2) % 4]], ssc[b2]).wait()

                @pl.when((g2 % 2) == c)
                def _drain_pull():
                    pltpu.make_async_copy(
                        rows[b2], pulled_hbm.at[pl.ds(st2, CHUNK)],
                        sp[b2]).wait()

            pltpu.async_copy(x_hbm.at[sb[bb]], rows[b2], sg[b2]).wait()

            @pl.when((g % 2) == c)
            def _pull():
                pltpu.async_copy(
                    rows[b2], pulled_hbm.at[pl.ds(st, CHUNK)], sp[b2])

            pltpu.async_copy(rows[b2], newemb_hbm.at[ib[bb]], ssc[b2])
        return carry

    lax.fori_loop(0, P2_QUADS, it2, 0)
    for iz in (P2_ITERS - 2, P2_ITERS - 1):
        b2 = iz % 2
        gz, stz = p2start_of(iz)
        pltpu.make_async_copy(
            rows[b2], newemb_hbm.at[ib[iz % 4]], ssc[b2]).wait()

        @pl.when((gz % 2) == c)
        def _drain_pull_z():
            pltpu.make_async_copy(
                rows[b2], pulled_hbm.at[pl.ds(stz, CHUNK)], sp[b2]).wait()
    scope2.__exit__(None, None, None)


def kernel(x, node_indices, embedding):
    idx32 = node_indices.astype(jnp.int32)
    f = pl.kernel(
        _body,
        out_type=(
            jax.ShapeDtypeStruct((N_NODES, D), jnp.float32),
            jax.ShapeDtypeStruct((B, D), jnp.float32),
            jax.ShapeDtypeStruct((B,), jnp.int32),
        ),
        mesh=plsc.VectorSubcoreMesh(core_axis_name="c", subcore_axis_name="s"),
        compiler_params=pltpu.CompilerParams(needs_layout_passes=False),
        scratch_types=[
            pltpu.VMEM((N_NODES,), jnp.int32),
            pltpu.VMEM((WCHUNK,), jnp.int32),
            pltpu.VMEM((WCHUNK,), jnp.int32),
            pltpu.VMEM((WCHUNK,), jnp.int32),
            pltpu.VMEM((WCHUNK,), jnp.int32),
            pltpu.VMEM((CHUNK, D), jnp.float32),
            pltpu.VMEM((CHUNK, D), jnp.float32),
            pltpu.VMEM((CHUNK,), jnp.int32),
            pltpu.VMEM((CHUNK,), jnp.int32),
            pltpu.VMEM((CHUNK,), jnp.int32),
            pltpu.VMEM((CHUNK,), jnp.int32),
            pltpu.VMEM((CHUNK,), jnp.int32),
            pltpu.VMEM((CHUNK,), jnp.int32),
            pltpu.VMEM((CHUNK,), jnp.int32),
            pltpu.VMEM((CHUNK,), jnp.int32),
            pltpu.SemaphoreType.DMA,
            pltpu.SemaphoreType.DMA,
            pltpu.SemaphoreType.DMA,
            pltpu.SemaphoreType.DMA,
            pltpu.SemaphoreType.DMA,
            pltpu.SemaphoreType.DMA,
            pltpu.SemaphoreType.DMA,
            pltpu.SemaphoreType.DMA,
            pltpu.SemaphoreType.DMA,
            pltpu.SemaphoreType.DMA,
        ],
    )
    new_emb, pulled, _ = f(x, idx32, embedding)
    return (new_emb, pulled)


# speedup vs baseline: 1.2738x; 1.2738x over previous
import jax
import jax.numpy as jnp
from jax import lax
from jax.experimental import pallas as pl
from jax.experimental.pallas import tpu as pltpu
from jax.experimental.pallas import tpu_sc as plsc

N_NODES = 100000
D = 128
B = 50000

L = 16
NSUB = 16

CHUNK = 80

N_COPY = N_NODES // CHUNK
COPY_LAST = N_NODES - CHUNK
N_CW = 2 * (NSUB - 2)
COPY_ITERS = -(-N_COPY // N_CW)
COPY_ITERS += COPY_ITERS % 2

N_P2 = B // CHUNK
P2_PER_CORE = -(-N_P2 // 2)
P2_ITERS = -(-P2_PER_CORE // NSUB)
P2_QUADS = P2_ITERS // 4

WCHUNK = 2000
N_WCH = B // WCHUNK
GRP = WCHUNK // L
GUNROLL = 5
WSPLIT = 13


def _body(x_hbm, idx_hbm, emb_hbm, newemb_hbm, pulled_hbm, src_hbm,
          w_v, idxc0, idxc1, srcc0, srcc1, rows0, rows1,
          ib0, ib1, ib2, ib3, sb0, sb1, sb2, sb3,
          sio0, sio1, sio2, sio3, sg0, sg1, ssc0, ssc1, sp0, sp1, xsem):
    c = lax.axis_index("c")
    s = lax.axis_index("s")
    idxc = (idxc0, idxc1)
    srcc = (srcc0, srcc1)
    rows = (rows0, rows1)
    ib = (ib0, ib1, ib2, ib3)
    sb = (sb0, sb1, sb2, sb3)
    sio = (sio0, sio1, sio2, sio3)
    sg = (sg0, sg1)
    ssc = (ssc0, ssc1)
    sp = (sp0, sp1)

    @pl.when(s < 2)
    @jax.named_scope("wpass")
    def _wpass():
        iota = lax.iota(jnp.int32, L)
        pltpu.async_copy(idx_hbm.at[pl.ds(0, WCHUNK)], idxc[0], sio[0])
        for ci in range(N_WCH):
            b = ci % 2
            if ci + 1 < N_WCH:
                pltpu.async_copy(
                    idx_hbm.at[pl.ds((ci + 1) * WCHUNK, WCHUNK)],
                    idxc[1 - b], sio[1 - b])
            pltpu.make_async_copy(
                idx_hbm.at[pl.ds(ci * WCHUNK, WCHUNK)], idxc[b], sio[b]).wait()
            cstart = ci * WCHUNK

            def grp(go, carry, _b=b, _cstart=cstart):
                for k in range(GUNROLL):
                    g = go * GUNROLL + k
                    idxv = idxc[_b][pl.ds(g * L, L)]
                    jv = (_cstart + g * L) + iota
                    _, keep = plsc.scan_count(idxv)
                    plsc.store_scatter(w_v, [idxv], jv, mask=keep)
                return carry

            lax.fori_loop(0, GRP // GUNROLL, grp, 0)
        lo = jnp.where(s == 0, 0, WSPLIT)
        hi = jnp.where(s == 0, WSPLIT, N_WCH)
        nch = jnp.int32(N_WCH)

        def gpass(t, carry):
            ci = lo + t
            live = ci < hi
            cin = jnp.minimum(ci, nch - 1)

            @pl.when(live)
            def _issue_in():
                pltpu.async_copy(
                    idx_hbm.at[pl.ds(cin * WCHUNK, WCHUNK)],
                    idxc[0], sio[0])
                pltpu.make_async_copy(
                    idx_hbm.at[pl.ds(cin * WCHUNK, WCHUNK)],
                    idxc[0], sio[0]).wait()

                def grp2(go, carry2):
                    for k in range(GUNROLL):
                        g = go * GUNROLL + k
                        idxv = idxc[0][pl.ds(g * L, L)]
                        srcc[0][pl.ds(g * L, L)] = plsc.load_gather(
                            w_v, [idxv])
                    return carry2

                lax.fori_loop(0, GRP // GUNROLL, grp2, 0)
                pltpu.async_copy(
                    srcc[0], src_hbm.at[pl.ds(cin * WCHUNK, WCHUNK)], ssc[0])
                pltpu.make_async_copy(
                    srcc[0], src_hbm.at[pl.ds(cin * WCHUNK, WCHUNK)],
                    ssc[0]).wait()
            return carry

        lax.fori_loop(0, WSPLIT, gpass, 0)

    @pl.when(s >= 2)
    @jax.named_scope("copy")
    def _copy():
        v = c * (NSUB - 2) + (s - 2)

        def cstart_of(j):
            g = jnp.minimum(v + N_CW * j, N_COPY - 1)
            return jnp.minimum(g * CHUNK, COPY_LAST)

        pltpu.async_copy(emb_hbm.at[pl.ds(cstart_of(0), CHUNK)], rows[0], sg[0])

        def it(jp, carry):
            for bb in range(2):
                j = jp * 2 + bb
                st = cstart_of(j)

                @pl.when(j + 1 < COPY_ITERS)
                def _prefetch():
                    @pl.when(j >= 1)
                    def _drain_out():
                        pltpu.make_async_copy(
                            rows[1 - bb],
                            newemb_hbm.at[pl.ds(cstart_of(j - 1), CHUNK)],
                            ssc[1 - bb]).wait()
                    pltpu.async_copy(
                        emb_hbm.at[pl.ds(cstart_of(j + 1), CHUNK)],
                        rows[1 - bb], sg[1 - bb])

                pltpu.make_async_copy(
                    emb_hbm.at[pl.ds(st, CHUNK)], rows[bb], sg[bb]).wait()
                pltpu.async_copy(
                    rows[bb], newemb_hbm.at[pl.ds(st, CHUNK)], ssc[bb])
            return carry

        lax.fori_loop(0, COPY_ITERS // 2, it, 0)
        for jz in (COPY_ITERS - 2, COPY_ITERS - 1):
            pltpu.make_async_copy(
                rows[jz % 2], newemb_hbm.at[pl.ds(cstart_of(jz), CHUNK)],
                ssc[jz % 2]).wait()

    plsc.subcore_barrier()

    @pl.when(s == 0)
    def _xcore():
        pltpu.semaphore_signal(xsem, 1, core_index=1 - c)
        pl.semaphore_wait(xsem, 1)

    plsc.subcore_barrier()

    scope2 = jax.named_scope("phase2")
    scope2.__enter__()
    me = P2_PER_CORE - c

    def p2start_of(i):
        gi = jnp.minimum(s + NSUB * i, me - 1)
        g = 2 * gi + c
        return g * CHUNK

    st0 = p2start_of(0)
    pltpu.async_copy(idx_hbm.at[pl.ds(st0, CHUNK)], ib[0], sio[0])
    pltpu.async_copy(src_hbm.at[pl.ds(st0, CHUNK)], sb[0], sio[0])

    def it2(q, carry):
        for bb in range(4):
            i = q * 4 + bb
            b2 = bb % 2
            st = p2start_of(i)

            @pl.when(i + 1 < P2_ITERS)
            def _prefetch():
                st1 = p2start_of(i + 1)
                pltpu.async_copy(
                    idx_hbm.at[pl.ds(st1, CHUNK)], ib[(bb + 1) % 4],
                    sio[(bb + 1) % 4])
                pltpu.async_copy(
                    src_hbm.at[pl.ds(st1, CHUNK)], sb[(bb + 1) % 4],
                    sio[(bb + 1) % 4])

            pltpu.make_async_copy(
                idx_hbm.at[pl.ds(st, CHUNK)], ib[bb], sio[bb]).wait()
            pltpu.make_async_copy(
                src_hbm.at[pl.ds(st, CHUNK)], sb[bb], sio[bb]).wait()

            @pl.when(i >= 2)
            def _drain_prev():
                st2 = p2start_of(i - 2)
                pltpu.make_async_copy(
                    rows[b2], newemb_hbm.at[ib[(bb + 2) % 4]], ssc[b2]).wait()
                pltpu.make_async_copy(
                    rows[b2], pulled_hbm.at[pl.ds(st2, CHUNK)], sp[b2]).wait()

            pltpu.async_copy(x_hbm.at[sb[bb]], rows[b2], sg[b2]).wait()
            pltpu.async_copy(
                rows[b2], pulled_hbm.at[pl.ds(st, CHUNK)], sp[b2])
            pltpu.async_copy(rows[b2], newemb_hbm.at[ib[bb]], ssc[b2])
        return carry

    lax.fori_loop(0, P2_QUADS, it2, 0)
    for iz in (P2_ITERS - 2, P2_ITERS - 1):
        b2 = iz % 2
        stz = p2start_of(iz)
        pltpu.make_async_copy(
            rows[b2], newemb_hbm.at[ib[iz % 4]], ssc[b2]).wait()
        pltpu.make_async_copy(
            rows[b2], pulled_hbm.at[pl.ds(stz, CHUNK)], sp[b2]).wait()
    scope2.__exit__(None, None, None)


def kernel(x, node_indices, embedding):
    idx32 = node_indices.astype(jnp.int32)
    f = pl.kernel(
        _body,
        out_type=(
            jax.ShapeDtypeStruct((N_NODES, D), jnp.float32),
            jax.ShapeDtypeStruct((B, D), jnp.float32),
            jax.ShapeDtypeStruct((B,), jnp.int32),
        ),
        mesh=plsc.VectorSubcoreMesh(core_axis_name="c", subcore_axis_name="s"),
        compiler_params=pltpu.CompilerParams(needs_layout_passes=False),
        scratch_types=[
            pltpu.VMEM((N_NODES,), jnp.int32),
            pltpu.VMEM((WCHUNK,), jnp.int32),
            pltpu.VMEM((WCHUNK,), jnp.int32),
            pltpu.VMEM((WCHUNK,), jnp.int32),
            pltpu.VMEM((WCHUNK,), jnp.int32),
            pltpu.VMEM((CHUNK, D), jnp.float32),
            pltpu.VMEM((CHUNK, D), jnp.float32),
            pltpu.VMEM((CHUNK,), jnp.int32),
            pltpu.VMEM((CHUNK,), jnp.int32),
            pltpu.VMEM((CHUNK,), jnp.int32),
            pltpu.VMEM((CHUNK,), jnp.int32),
            pltpu.VMEM((CHUNK,), jnp.int32),
            pltpu.VMEM((CHUNK,), jnp.int32),
            pltpu.VMEM((CHUNK,), jnp.int32),
            pltpu.VMEM((CHUNK,), jnp.int32),
            pltpu.SemaphoreType.DMA,
            pltpu.SemaphoreType.DMA,
            pltpu.SemaphoreType.DMA,
            pltpu.SemaphoreType.DMA,
            pltpu.SemaphoreType.DMA,
            pltpu.SemaphoreType.DMA,
            pltpu.SemaphoreType.DMA,
            pltpu.SemaphoreType.DMA,
            pltpu.SemaphoreType.DMA,
            pltpu.SemaphoreType.DMA,
            pltpu.SemaphoreType.REGULAR,
        ],
    )
    new_emb, pulled, _ = f(x, idx32, embedding)
    return (new_emb, pulled)


# speedup vs baseline: 1.4178x; 1.1131x over previous
import jax
import jax.numpy as jnp
from jax import lax
from jax.experimental import pallas as pl
from jax.experimental.pallas import tpu as pltpu
from jax.experimental.pallas import tpu_sc as plsc

N_NODES = 100000
D = 128
B = 50000

L = 16
NSUB = 16

CHUNK = 80

N_COPY = N_NODES // CHUNK
COPY_LAST = N_NODES - CHUNK
N_CW = 2 * (NSUB - 2)
COPY_ITERS = -(-N_COPY // N_CW)
COPY_ITERS += COPY_ITERS % 2

N_P2 = B // CHUNK
P2_PER_CORE = -(-N_P2 // 2)
P2_ITERS = -(-P2_PER_CORE // NSUB)
P2_QUADS = P2_ITERS // 4

WCHUNK = 2000
N_WCH = B // WCHUNK
GRP = WCHUNK // L
GUNROLL = 5
WSPLIT = 13


def _body(x_hbm, idx_hbm, emb_hbm, newemb_hbm, pulled_hbm, src_hbm,
          w_v, idxc0, idxc1, srcc0, srcc1, rows0, rows1,
          ib0, ib1, ib2, ib3, sb0, sb1, sb2, sb3,
          sio0, sio1, sio2, sio3, sg0, sg1, ssc0, ssc1, sp0, sp1, xsem):
    c = lax.axis_index("c")
    s = lax.axis_index("s")
    idxc = (idxc0, idxc1)
    srcc = (srcc0, srcc1)
    rows = (rows0, rows1)
    ib = (ib0, ib1, ib2, ib3)
    sb = (sb0, sb1, sb2, sb3)
    sio = (sio0, sio1, sio2, sio3)
    sg = (sg0, sg1)
    ssc = (ssc0, ssc1)
    sp = (sp0, sp1)

    @pl.when(s < 2)
    @jax.named_scope("wpass")
    def _wpass():
        iota = lax.iota(jnp.int32, L)
        pltpu.async_copy(idx_hbm.at[pl.ds(0, WCHUNK)], idxc[0], sio[0])
        for ci in range(N_WCH):
            b = ci % 2
            if ci + 1 < N_WCH:
                pltpu.async_copy(
                    idx_hbm.at[pl.ds((ci + 1) * WCHUNK, WCHUNK)],
                    idxc[1 - b], sio[1 - b])
            pltpu.make_async_copy(
                idx_hbm.at[pl.ds(ci * WCHUNK, WCHUNK)], idxc[b], sio[b]).wait()
            cstart = ci * WCHUNK

            def grp(go, carry, _b=b, _cstart=cstart):
                for k in range(GUNROLL):
                    g = go * GUNROLL + k
                    idxv = idxc[_b][pl.ds(g * L, L)]
                    jv = (_cstart + g * L) + iota
                    plsc.store_scatter(w_v, [idxv], jv)
                return carry

            lax.fori_loop(0, GRP // GUNROLL, grp, 0)
        lo = jnp.where(s == 0, 0, WSPLIT)

        def ci_of(t):
            return jnp.minimum(lo + t, N_WCH - 1)

        pltpu.async_copy(
            idx_hbm.at[pl.ds(ci_of(0) * WCHUNK, WCHUNK)], idxc[0], sio[0])
        for t in range(WSPLIT):
            b = t % 2
            if t + 1 < WSPLIT:
                pltpu.async_copy(
                    idx_hbm.at[pl.ds(ci_of(t + 1) * WCHUNK, WCHUNK)],
                    idxc[1 - b], sio[1 - b])
            pltpu.make_async_copy(
                idx_hbm.at[pl.ds(ci_of(t) * WCHUNK, WCHUNK)],
                idxc[b], sio[b]).wait()
            if t >= 2:
                pltpu.make_async_copy(
                    srcc[b], src_hbm.at[pl.ds(ci_of(t - 2) * WCHUNK, WCHUNK)],
                    ssc[b]).wait()

            def grp2(go, carry2, _b=b):
                for k in range(GUNROLL):
                    g = go * GUNROLL + k
                    idxv = idxc[_b][pl.ds(g * L, L)]
                    srcc[_b][pl.ds(g * L, L)] = plsc.load_gather(w_v, [idxv])
                return carry2

            lax.fori_loop(0, GRP // GUNROLL, grp2, 0)
            pltpu.async_copy(
                srcc[b], src_hbm.at[pl.ds(ci_of(t) * WCHUNK, WCHUNK)], ssc[b])
        for t in (WSPLIT - 2, WSPLIT - 1):
            pltpu.make_async_copy(
                srcc[t % 2], src_hbm.at[pl.ds(ci_of(t) * WCHUNK, WCHUNK)],
                ssc[t % 2]).wait()

    @pl.when(s >= 2)
    @jax.named_scope("copy")
    def _copy():
        v = c * (NSUB - 2) + (s - 2)

        def cstart_of(j):
            g = jnp.minimum(v + N_CW * j, N_COPY - 1)
            return jnp.minimum(g * CHUNK, COPY_LAST)

        pltpu.async_copy(emb_hbm.at[pl.ds(cstart_of(0), CHUNK)], rows[0], sg[0])

        def it(jp, carry):
            for bb in range(2):
                j = jp * 2 + bb
                st = cstart_of(j)

                @pl.when(j + 1 < COPY_ITERS)
                def _prefetch():
                    @pl.when(j >= 1)
                    def _drain_out():
                        pltpu.make_async_copy(
                            rows[1 - bb],
                            newemb_hbm.at[pl.ds(cstart_of(j - 1), CHUNK)],
                            ssc[1 - bb]).wait()
                    pltpu.async_copy(
                        emb_hbm.at[pl.ds(cstart_of(j + 1), CHUNK)],
                        rows[1 - bb], sg[1 - bb])

                pltpu.make_async_copy(
                    emb_hbm.at[pl.ds(st, CHUNK)], rows[bb], sg[bb]).wait()
                pltpu.async_copy(
                    rows[bb], newemb_hbm.at[pl.ds(st, CHUNK)], ssc[bb])
            return carry

        lax.fori_loop(0, COPY_ITERS // 2, it, 0)
        for jz in (COPY_ITERS - 2, COPY_ITERS - 1):
            pltpu.make_async_copy(
                rows[jz % 2], newemb_hbm.at[pl.ds(cstart_of(jz), CHUNK)],
                ssc[jz % 2]).wait()

    plsc.subcore_barrier()

    @pl.when(s == 0)
    def _xcore():
        pltpu.semaphore_signal(xsem, 1, core_index=1 - c)
        pl.semaphore_wait(xsem, 1)

    plsc.subcore_barrier()

    scope2 = jax.named_scope("phase2")
    scope2.__enter__()
    me = P2_PER_CORE - c

    def p2start_of(i):
        gi = jnp.minimum(s + NSUB * i, me - 1)
        g = 2 * gi + c
        return g * CHUNK

    st0 = p2start_of(0)
    pltpu.async_copy(idx_hbm.at[pl.ds(st0, CHUNK)], ib[0], sio[0])
    pltpu.async_copy(src_hbm.at[pl.ds(st0, CHUNK)], sb[0], sio[0])

    def it2(q, carry):
        for bb in range(4):
            i = q * 4 + bb
            b2 = bb % 2
            st = p2start_of(i)

            @pl.when(i + 1 < P2_ITERS)
            def _prefetch():
                st1 = p2start_of(i + 1)
                pltpu.async_copy(
                    idx_hbm.at[pl.ds(st1, CHUNK)], ib[(bb + 1) % 4],
                    sio[(bb + 1) % 4])
                pltpu.async_copy(
                    src_hbm.at[pl.ds(st1, CHUNK)], sb[(bb + 1) % 4],
                    sio[(bb + 1) % 4])

            pltpu.make_async_copy(
                idx_hbm.at[pl.ds(st, CHUNK)], ib[bb], sio[bb]).wait()
            pltpu.make_async_copy(
                src_hbm.at[pl.ds(st, CHUNK)], sb[bb], sio[bb]).wait()

            @pl.when(i >= 2)
            def _drain_prev():
                st2 = p2start_of(i - 2)
                pltpu.make_async_copy(
                    rows[b2], newemb_hbm.at[ib[(bb + 2) % 4]], ssc[b2]).wait()
                pltpu.make_async_copy(
                    rows[b2], pulled_hbm.at[pl.ds(st2, CHUNK)], sp[b2]).wait()

            pltpu.async_copy(x_hbm.at[sb[bb]], rows[b2], sg[b2]).wait()
            pltpu.async_copy(
                rows[b2], pulled_hbm.at[pl.ds(st, CHUNK)], sp[b2])
            pltpu.async_copy(rows[b2], newemb_hbm.at[ib[bb]], ssc[b2])
        return carry

    lax.fori_loop(0, P2_QUADS, it2, 0)
    for iz in (P2_ITERS - 2, P2_ITERS - 1):
        b2 = iz % 2
        stz = p2start_of(iz)
        pltpu.make_async_copy(
            rows[b2], newemb_hbm.at[ib[iz % 4]], ssc[b2]).wait()
        pltpu.make_async_copy(
            rows[b2], pulled_hbm.at[pl.ds(stz, CHUNK)], sp[b2]).wait()
    scope2.__exit__(None, None, None)


def kernel(x, node_indices, embedding):
    idx32 = node_indices.astype(jnp.int32)
    f = pl.kernel(
        _body,
        out_type=(
            jax.ShapeDtypeStruct((N_NODES, D), jnp.float32),
            jax.ShapeDtypeStruct((B, D), jnp.float32),
            jax.ShapeDtypeStruct((B,), jnp.int32),
        ),
        mesh=plsc.VectorSubcoreMesh(core_axis_name="c", subcore_axis_name="s"),
        compiler_params=pltpu.CompilerParams(needs_layout_passes=False),
        scratch_types=[
            pltpu.VMEM((N_NODES,), jnp.int32),
            pltpu.VMEM((WCHUNK,), jnp.int32),
            pltpu.VMEM((WCHUNK,), jnp.int32),
            pltpu.VMEM((WCHUNK,), jnp.int32),
            pltpu.VMEM((WCHUNK,), jnp.int32),
            pltpu.VMEM((CHUNK, D), jnp.float32),
            pltpu.VMEM((CHUNK, D), jnp.float32),
            pltpu.VMEM((CHUNK,), jnp.int32),
            pltpu.VMEM((CHUNK,), jnp.int32),
            pltpu.VMEM((CHUNK,), jnp.int32),
            pltpu.VMEM((CHUNK,), jnp.int32),
            pltpu.VMEM((CHUNK,), jnp.int32),
            pltpu.VMEM((CHUNK,), jnp.int32),
            pltpu.VMEM((CHUNK,), jnp.int32),
            pltpu.VMEM((CHUNK,), jnp.int32),
            pltpu.SemaphoreType.DMA,
            pltpu.SemaphoreType.DMA,
            pltpu.SemaphoreType.DMA,
            pltpu.SemaphoreType.DMA,
            pltpu.SemaphoreType.DMA,
            pltpu.SemaphoreType.DMA,
            pltpu.SemaphoreType.DMA,
            pltpu.SemaphoreType.DMA,
            pltpu.SemaphoreType.DMA,
            pltpu.SemaphoreType.DMA,
            pltpu.SemaphoreType.REGULAR,
        ],
    )
    new_emb, pulled, _ = f(x, idx32, embedding)
    return (new_emb, pulled)


# speedup vs baseline: 1.5581x; 1.0990x over previous
import jax
import jax.numpy as jnp
from jax import lax
from jax.experimental import pallas as pl
from jax.experimental.pallas import tpu as pltpu
from jax.experimental.pallas import tpu_sc as plsc

N_NODES = 100000
D = 128
B = 50000

L = 16
NSUB = 16

CHUNK = 128
ROWS_ARENA = 782

N_COPY = -(-N_NODES // CHUNK)
COPY_LAST = N_NODES - CHUNK
N_CW = 2 * (NSUB - 2)
COPY_ITERS = -(-N_COPY // N_CW)

N_P2 = -(-B // CHUNK)
P2_LAST = B - CHUNK
P2_ITERS = 13

WCHUNK = 2000
N_WCH = B // WCHUNK
GRP = WCHUNK // L
GUNROLL = 5
WSPLIT = 13


def _body(x_hbm, idx_hbm, emb_hbm, newemb_hbm, pulled_hbm, src_hbm,
          arena, idxc0, idxc1, srcc0, srcc1,
          ib0, ib1, ib2, ib3, sb0, sb1, sb2, sb3,
          sio0, sio1, sio2, sio3, sg0, sg1, sg2, sg3,
          ssc0, ssc1, ssc2, ssc3, sp0, sp1, sp2, sp3, xsem):
    c = lax.axis_index("c")
    s = lax.axis_index("s")
    idxc = (idxc0, idxc1)
    srcc = (srcc0, srcc1)
    ib = (ib0, ib1, ib2, ib3)
    sb = (sb0, sb1, sb2, sb3)
    sio = (sio0, sio1, sio2, sio3)
    sg = (sg0, sg1, sg2, sg3)
    ssc = (ssc0, ssc1, ssc2, ssc3)
    sp = (sp0, sp1, sp2, sp3)
    rows = tuple(arena.at[pl.ds(k * CHUNK, CHUNK)] for k in range(4))

    @pl.when(s < 2)
    @jax.named_scope("wpass")
    def _wpass():
        iota = lax.iota(jnp.int32, L)
        pltpu.async_copy(idx_hbm.at[pl.ds(0, WCHUNK)], idxc[0], sio[0])
        for ci in range(N_WCH):
            b = ci % 2
            if ci + 1 < N_WCH:
                pltpu.async_copy(
                    idx_hbm.at[pl.ds((ci + 1) * WCHUNK, WCHUNK)],
                    idxc[1 - b], sio[1 - b])
            pltpu.make_async_copy(
                idx_hbm.at[pl.ds(ci * WCHUNK, WCHUNK)], idxc[b], sio[b]).wait()
            cstart = ci * WCHUNK

            def grp(go, carry, _b=b, _cstart=cstart):
                for k in range(GUNROLL):
                    g = go * GUNROLL + k
                    idxv = idxc[_b][pl.ds(g * L, L)]
                    jv = (_cstart + g * L) + iota
                    rowv = lax.shift_right_logical(idxv, 7)
                    colv = lax.bitwise_and(idxv, 127)
                    plsc.store_scatter(
                        arena, [rowv, colv], plsc.bitcast(jv, jnp.float32))
                return carry

            lax.fori_loop(0, GRP // GUNROLL, grp, 0)
        lo = jnp.where(s == 0, 0, WSPLIT)

        def ci_of(t):
            return jnp.minimum(lo + t, N_WCH - 1)

        pltpu.async_copy(
            idx_hbm.at[pl.ds(ci_of(0) * WCHUNK, WCHUNK)], idxc[0], sio[0])
        for t in range(WSPLIT):
            b = t % 2
            if t + 1 < WSPLIT:
                pltpu.async_copy(
                    idx_hbm.at[pl.ds(ci_of(t + 1) * WCHUNK, WCHUNK)],
                    idxc[1 - b], sio[1 - b])
            pltpu.make_async_copy(
                idx_hbm.at[pl.ds(ci_of(t) * WCHUNK, WCHUNK)],
                idxc[b], sio[b]).wait()
            if t >= 2:
                pltpu.make_async_copy(
                    srcc[b], src_hbm.at[pl.ds(ci_of(t - 2) * WCHUNK, WCHUNK)],
                    ssc[b]).wait()

            def grp2(go, carry2, _b=b):
                for k in range(GUNROLL):
                    g = go * GUNROLL + k
                    idxv = idxc[_b][pl.ds(g * L, L)]
                    rowv = lax.shift_right_logical(idxv, 7)
                    colv = lax.bitwise_and(idxv, 127)
                    srcc[_b][pl.ds(g * L, L)] = plsc.bitcast(
                        plsc.load_gather(arena, [rowv, colv]), jnp.int32)
                return carry2

            lax.fori_loop(0, GRP // GUNROLL, grp2, 0)
            pltpu.async_copy(
                srcc[b], src_hbm.at[pl.ds(ci_of(t) * WCHUNK, WCHUNK)], ssc[b])
        for t in (WSPLIT - 2, WSPLIT - 1):
            pltpu.make_async_copy(
                srcc[t % 2], src_hbm.at[pl.ds(ci_of(t) * WCHUNK, WCHUNK)],
                ssc[t % 2]).wait()

    @pl.when(s >= 2)
    @jax.named_scope("copy")
    def _copy():
        v = c * (NSUB - 2) + (s - 2)

        def cstart_of(j):
            g = jnp.minimum(v + N_CW * j, N_COPY - 1)
            return jnp.minimum(g * CHUNK, COPY_LAST)

        for j0 in range(2):
            pltpu.async_copy(
                emb_hbm.at[pl.ds(cstart_of(j0), CHUNK)], rows[j0], sg[j0])

        def it(jq, carry):
            for kk in range(4):
                j = jq * 4 + kk

                @pl.when(j >= 2)
                def _drain_out():
                    pltpu.make_async_copy(
                        rows[(kk + 2) % 4],
                        newemb_hbm.at[pl.ds(cstart_of(j - 2), CHUNK)],
                        ssc[(kk + 2) % 4]).wait()

                @pl.when(j + 2 < COPY_ITERS)
                def _prefetch():
                    pltpu.async_copy(
                        emb_hbm.at[pl.ds(cstart_of(j + 2), CHUNK)],
                        rows[(kk + 2) % 4], sg[(kk + 2) % 4])

                pltpu.make_async_copy(
                    emb_hbm.at[pl.ds(cstart_of(j), CHUNK)], rows[kk],
                    sg[kk]).wait()
                pltpu.async_copy(
                    rows[kk], newemb_hbm.at[pl.ds(cstart_of(j), CHUNK)],
                    ssc[kk])
            return carry

        lax.fori_loop(0, COPY_ITERS // 4, it, 0)
        for jz in (COPY_ITERS - 2, COPY_ITERS - 1):
            pltpu.make_async_copy(
                rows[jz % 4], newemb_hbm.at[pl.ds(cstart_of(jz), CHUNK)],
                ssc[jz % 4]).wait()

    plsc.subcore_barrier()

    @pl.when(s == 0)
    def _xcore():
        pltpu.semaphore_signal(xsem, 1, core_index=1 - c)
        pl.semaphore_wait(xsem, 1)

    plsc.subcore_barrier()

    scope2 = jax.named_scope("phase2")
    scope2.__enter__()
    me = -(-N_P2 // 2) - c

    def p2start_of(i):
        gi = jnp.minimum(s + NSUB * i, me - 1)
        g = 2 * gi + c
        return jnp.minimum(g * CHUNK, P2_LAST)

    def p2_io_start(i, slot):
        st = p2start_of(i)
        pltpu.async_copy(idx_hbm.at[pl.ds(st, CHUNK)], ib[slot], sio[slot])
        pltpu.async_copy(src_hbm.at[pl.ds(st, CHUNK)], sb[slot], sio[slot])

    def p2_io_wait(i, slot):
        st = p2start_of(i)
        pltpu.make_async_copy(
            idx_hbm.at[pl.ds(st, CHUNK)], ib[slot], sio[slot]).wait()
        pltpu.make_async_copy(
            src_hbm.at[pl.ds(st, CHUNK)], sb[slot], sio[slot]).wait()

    def p2_drain(i, slot):
        st = p2start_of(i)
        pltpu.make_async_copy(
            rows[slot], newemb_hbm.at[ib[slot]], ssc[slot]).wait()
        pltpu.make_async_copy(
            rows[slot], pulled_hbm.at[pl.ds(st, CHUNK)], sp[slot]).wait()

    def p2_main(i, slot):
        st = p2start_of(i)
        pltpu.make_async_copy(x_hbm.at[sb[slot]], rows[slot], sg[slot]).wait()
        pltpu.async_copy(rows[slot], pulled_hbm.at[pl.ds(st, CHUNK)], sp[slot])
        pltpu.async_copy(rows[slot], newemb_hbm.at[ib[slot]], ssc[slot])

    p2_io_start(0, 0)
    p2_io_start(1, 1)
    p2_io_wait(0, 0)
    pltpu.async_copy(x_hbm.at[sb[0]], rows[0], sg[0])

    def it2(q, carry):
        for kk in range(4):
            i = q * 4 + kk

            @pl.when(i >= 2)
            def _a():
                p2_drain(i - 2, (kk + 2) % 4)

            @pl.when(i + 1 < P2_ITERS)
            def _bc():
                p2_io_wait(i + 1, (kk + 1) % 4)
                pltpu.async_copy(
                    x_hbm.at[sb[(kk + 1) % 4]], rows[(kk + 1) % 4],
                    sg[(kk + 1) % 4])

            @pl.when(i + 2 < P2_ITERS)
            def _d():
                p2_io_start(i + 2, (kk + 2) % 4)

            p2_main(i, kk)
        return carry

    lax.fori_loop(0, P2_ITERS // 4, it2, 0)
    for i in range(4 * (P2_ITERS // 4), P2_ITERS):
        kk = i % 4
        if i >= 2:
            p2_drain(i - 2, (kk + 2) % 4)
        if i + 1 < P2_ITERS:
            p2_io_wait(i + 1, (kk + 1) % 4)
            pltpu.async_copy(
                x_hbm.at[sb[(kk + 1) % 4]], rows[(kk + 1) % 4],
                sg[(kk + 1) % 4])
        if i + 2 < P2_ITERS:
            p2_io_start(i + 2, (kk + 2) % 4)
        p2_main(i, kk)
    for i in (P2_ITERS - 2, P2_ITERS - 1):
        p2_drain(i, i % 4)
    scope2.__exit__(None, None, None)


def kernel(x, node_indices, embedding):
    idx32 = node_indices.astype(jnp.int32)
    f = pl.kernel(
        _body,
        out_type=(
            jax.ShapeDtypeStruct((N_NODES, D), jnp.float32),
            jax.ShapeDtypeStruct((B, D), jnp.float32),
            jax.ShapeDtypeStruct((B,), jnp.int32),
        ),
        mesh=plsc.VectorSubcoreMesh(core_axis_name="c", subcore_axis_name="s"),
        compiler_params=pltpu.CompilerParams(needs_layout_passes=False),
        scratch_types=[
            pltpu.VMEM((ROWS_ARENA, D), jnp.float32),
            pltpu.VMEM((WCHUNK,), jnp.int32),
            pltpu.VMEM((WCHUNK,), jnp.int32),
            pltpu.VMEM((WCHUNK,), jnp.int32),
            pltpu.VMEM((WCHUNK,), jnp.int32),
            pltpu.VMEM((CHUNK,), jnp.int32),
            pltpu.VMEM((CHUNK,), jnp.int32),
            pltpu.VMEM((CHUNK,), jnp.int32),
            pltpu.VMEM((CHUNK,), jnp.int32),
            pltpu.VMEM((CHUNK,), jnp.int32),
            pltpu.VMEM((CHUNK,), jnp.int32),
            pltpu.VMEM((CHUNK,), jnp.int32),
            pltpu.VMEM((CHUNK,), jnp.int32),
            pltpu.SemaphoreType.DMA,
            pltpu.SemaphoreType.DMA,
            pltpu.SemaphoreType.DMA,
            pltpu.SemaphoreType.DMA,
            pltpu.SemaphoreType.DMA,
            pltpu.SemaphoreType.DMA,
            pltpu.SemaphoreType.DMA,
            pltpu.SemaphoreType.DMA,
            pltpu.SemaphoreType.DMA,
            pltpu.SemaphoreType.DMA,
            pltpu.SemaphoreType.DMA,
            pltpu.SemaphoreType.DMA,
            pltpu.SemaphoreType.DMA,
            pltpu.SemaphoreType.DMA,
            pltpu.SemaphoreType.DMA,
            pltpu.SemaphoreType.DMA,
            pltpu.SemaphoreType.REGULAR,
        ],
    )
    new_emb, pulled, _ = f(x, idx32, embedding)
    return (new_emb, pulled)
